# baseline (speedup 1.0000x reference)
import functools

import jax
import jax.numpy as jnp
from jax.experimental import pallas as pl
from jax.experimental.pallas import tpu as pltpu

N0_, N1_ = 50000, 25000
DH = 64
BLK = 5000


def _matmul_body(x_ref, w_ref, y_ref):
    y_ref[...] = jnp.dot(x_ref[...], w_ref[...],
                         preferred_element_type=jnp.float32)


def _matmul(x, w, n, kdim, mdim):
    return pl.pallas_call(
        _matmul_body,
        grid=(n // BLK,),
        in_specs=[
            pl.BlockSpec((BLK, kdim), lambda i: (i, 0)),
            pl.BlockSpec((kdim, mdim), lambda i: (0, 0)),
        ],
        out_specs=pl.BlockSpec((BLK, mdim), lambda i: (i, 0)),
        out_shape=jax.ShapeDtypeStruct((n, mdim), jnp.float32),
    )(x, w)


def _colsum_body(y_ref, s_ref):
    s_ref[...] = jnp.sum(y_ref[...], axis=0, keepdims=True)[None]


def _colsum(y, n):
    nb = n // BLK
    return pl.pallas_call(
        _colsum_body,
        grid=(nb,),
        in_specs=[pl.BlockSpec((BLK, DH), lambda i: (i, 0))],
        out_specs=pl.BlockSpec((1, 1, DH), lambda i: (i, 0, 0)),
        out_shape=jax.ShapeDtypeStruct((nb, 1, DH), jnp.float32),
    )(y)


def _varpart_body(y_ref, mu_ref, s_ref):
    t = y_ref[...] - mu_ref[...]
    s_ref[...] = jnp.sum(t * t, axis=0, keepdims=True)[None]


def _varpart(y, mu, n):
    nb = n // BLK
    return pl.pallas_call(
        _varpart_body,
        grid=(nb,),
        in_specs=[
            pl.BlockSpec((BLK, DH), lambda i: (i, 0)),
            pl.BlockSpec((1, DH), lambda i: (0, 0)),
        ],
        out_specs=pl.BlockSpec((1, 1, DH), lambda i: (i, 0, 0)),
        out_shape=jax.ShapeDtypeStruct((nb, 1, DH), jnp.float32),
    )(y, mu)


def _bn_apply_body(y_ref, mu_ref, var_ref, g_ref, b_ref, o_ref, *, relu):
    o = (g_ref[...] * (y_ref[...] - mu_ref[...])
         / jnp.sqrt(var_ref[...] + 1e-5) + b_ref[...])
    if relu:
        o = jnp.maximum(o, 0.0)
    o_ref[...] = o


def _bn_apply(y, mu, var, g, b, n, relu):
    return pl.pallas_call(
        functools.partial(_bn_apply_body, relu=relu),
        grid=(n // BLK,),
        in_specs=[
            pl.BlockSpec((BLK, DH), lambda i: (i, 0)),
            pl.BlockSpec((1, DH), lambda i: (0, 0)),
            pl.BlockSpec((1, DH), lambda i: (0, 0)),
            pl.BlockSpec((1, DH), lambda i: (0, 0)),
            pl.BlockSpec((1, DH), lambda i: (0, 0)),
        ],
        out_specs=pl.BlockSpec((BLK, DH), lambda i: (i, 0)),
        out_shape=jax.ShapeDtypeStruct((n, DH), jnp.float32),
    )(y, mu, var, g.reshape(1, DH), b.reshape(1, DH))


def _batch_norm_opt(y, g, b, n, relu):
    nb = n // BLK
    s = _colsum(y, n)
    mu = (jnp.sum(s, axis=(0, 1)) / n).reshape(1, DH)
    vs = _varpart(y, mu, n)
    var = (jnp.sum(vs, axis=(0, 1)) / n).reshape(1, DH)
    return _bn_apply(y, mu, var, g, b, n, relu)


def _mp_dense_body(h_ref, agg_ref, deg_ref, w_ref, b_ref, o_ref, *, fuse_elu):
    h = h_ref[...]
    z = h + agg_ref[...] / deg_ref[...]
    z = jnp.dot(z, w_ref[...], preferred_element_type=jnp.float32) + b_ref[...]
    if fuse_elu:
        o_ref[...] = h + jnp.where(z > 0, z, (jnp.exp(z) - 1.0))
    else:
        o_ref[...] = z


def _mp_dense(h, agg, deg, w, b, n, fuse_elu=True):
    return pl.pallas_call(
        functools.partial(_mp_dense_body, fuse_elu=fuse_elu),
        grid=(n // BLK,),
        in_specs=[
            pl.BlockSpec((BLK, DH), lambda i: (i, 0)),
            pl.BlockSpec((BLK, DH), lambda i: (i, 0)),
            pl.BlockSpec((BLK, 1), lambda i: (i, 0)),
            pl.BlockSpec((DH, DH), lambda i: (0, 0)),
            pl.BlockSpec((1, DH), lambda i: (0, 0)),
        ],
        out_specs=pl.BlockSpec((BLK, DH), lambda i: (i, 0)),
        out_shape=jax.ShapeDtypeStruct((n, DH), jnp.float32),
    )(h, agg, deg, w, b.reshape(1, DH))


def _post_body(h_ref, hu_ref, w0, w1, w2, b0, b1, b2, y_ref, s_ref):
    x = h_ref[...] + hu_ref[...]

    def lin_elu(x, w, b):
        z = jnp.dot(x, w[...], preferred_element_type=jnp.float32) + b[...]
        return jnp.where(z > 0, z, (jnp.exp(z) - 1.0))

    x = lin_elu(x, w0, b0)
    x = lin_elu(x, w1, b1)
    x = lin_elu(x, w2, b2)
    y_ref[...] = x
    s_ref[...] = jnp.sum(x, axis=0, keepdims=True)[None]


def _post(h0, hu2, W_post, b_post, n):
    nb = n // BLK
    ws = [W_post[i] for i in range(3)]
    bs = [b_post[i].reshape(1, DH) for i in range(3)]
    return pl.pallas_call(
        _post_body,
        grid=(nb,),
        in_specs=[
            pl.BlockSpec((BLK, DH), lambda i: (i, 0)),
            pl.BlockSpec((BLK, DH), lambda i: (i, 0)),
        ]
        + [pl.BlockSpec((DH, DH), lambda i: (0, 0))] * 3
        + [pl.BlockSpec((1, DH), lambda i: (0, 0))] * 3,
        out_specs=[
            pl.BlockSpec((BLK, DH), lambda i: (i, 0)),
            pl.BlockSpec((1, 1, DH), lambda i: (i, 0, 0)),
        ],
        out_shape=[
            jax.ShapeDtypeStruct((n, DH), jnp.float32),
            jax.ShapeDtypeStruct((nb, 1, DH), jnp.float32),
        ],
    )(h0, hu2, *ws, *bs)


def _final_body(y_ref, mu_ref, var_ref, g_ref, b_ref, wc_ref, o_ref):
    x = (g_ref[...] * (y_ref[...] - mu_ref[...])
         / jnp.sqrt(var_ref[...] + 1e-5) + b_ref[...])
    o_ref[...] = jnp.dot(x, wc_ref[...], preferred_element_type=jnp.float32)


def _final(y, mu, var, g, b, wc_pad, n, ncls_pad):
    return pl.pallas_call(
        _final_body,
        grid=(n // BLK,),
        in_specs=[
            pl.BlockSpec((BLK, DH), lambda i: (i, 0)),
            pl.BlockSpec((1, DH), lambda i: (0, 0)),
            pl.BlockSpec((1, DH), lambda i: (0, 0)),
            pl.BlockSpec((1, DH), lambda i: (0, 0)),
            pl.BlockSpec((1, DH), lambda i: (0, 0)),
            pl.BlockSpec((DH, ncls_pad), lambda i: (0, 0)),
        ],
        out_specs=pl.BlockSpec((BLK, ncls_pad), lambda i: (i, 0)),
        out_shape=jax.ShapeDtypeStruct((n, ncls_pad), jnp.float32),
    )(y, mu, var, g.reshape(1, DH), b.reshape(1, DH), wc_pad)


def _seg_sum(h, src, dst, n):
    return jax.ops.segment_sum(h[src], dst, num_segments=n)


def _deg(dst, n):
    deg = jax.ops.segment_sum(jnp.ones(dst.shape, jnp.float32), dst,
                              num_segments=n)
    return jnp.maximum(deg, 1.0).reshape(n, 1)


def _mp_block_opt(h, src, dst, Ws, bs, n, exact=False):
    deg = _deg(dst, n)
    for l in range(Ws.shape[0]):
        agg = _seg_sum(h, src, dst, n)
        if exact:
            z = _mp_dense(h, agg, deg, Ws[l], bs[l], n, fuse_elu=False)
            h = h + jax.nn.elu(z)
        else:
            h = _mp_dense(h, agg, deg, Ws[l], bs[l], n)
    return h


def _push_up(h_coarse, idx, src_j, dst_i, n_fine):
    new_h = jnp.zeros((n_fine, DH), jnp.float32).at[idx].set(h_coarse)
    agg = jax.ops.segment_max(new_h[src_j], dst_i, num_segments=n_fine)
    return jnp.where(jnp.isfinite(agg), agg, 0.0)


def kernel(coord, feat, edge_index0, edge_index1, edge_index2, W_embed, g_embed,
           b_embed, W_mp0, b_mp0, W_mp1, b_mp1, W_bneck, b_bneck, w_score0,
           w_score1, W_up0, b_up0, W_up1, b_up1, W_post, b_post, g_final,
           b_final, W_cluster):
    del coord, edge_index2, W_bneck, b_bneck, w_score1, W_up0, b_up0

    y = feat @ W_embed
    mu0 = jnp.mean(y, axis=0, keepdims=True)
    var0 = jnp.var(y, axis=0, keepdims=True)
    h0 = jax.nn.relu(g_embed * (y - mu0) / jnp.sqrt(var0 + 1e-5) + b_embed)

    e0s, e0d = edge_index0[0], edge_index0[1]
    e1s, e1d = edge_index1[0], edge_index1[1]

    h0 = _mp_block_opt(h0, e0s, e0d, W_mp0, b_mp0, N0_, exact=True)

    v0, idx0 = jax.lax.top_k(h0 @ w_score0, N1_)
    h1 = h0[idx0] * jax.nn.sigmoid(v0)[:, None]

    h1 = _mp_block_opt(h1, e1s, e1d, W_mp1, b_mp1, N1_)

    hu2 = _push_up(h1, idx0, e0d, e0s, N0_)
    hu2 = _mp_block_opt(hu2, e0s, e0d, W_up1, b_up1, N0_)

    y3, _ = _post(h0, hu2, W_post, b_post, N0_)
    mu3 = jnp.mean(y3, axis=0, keepdims=True)
    var3 = jnp.var(y3, axis=0, keepdims=True)
    x = g_final * (y3 - mu3) / jnp.sqrt(var3 + 1e-5) + b_final
    return x @ W_cluster


# speedup vs baseline: 1.2584x; 1.2584x over previous
import functools

import jax
import jax.numpy as jnp
from jax import lax
from jax.experimental import pallas as pl
from jax.experimental.pallas import tpu as pltpu
from jax.experimental.pallas import tpu_sc as plsc

N0_, N1_ = 50000, 25000
DH = 64
BLK = 5000


def _matmul_body(x_ref, w_ref, y_ref):
    y_ref[...] = jnp.dot(x_ref[...], w_ref[...],
                         preferred_element_type=jnp.float32)


def _matmul(x, w, n, kdim, mdim):
    return pl.pallas_call(
        _matmul_body,
        grid=(n // BLK,),
        in_specs=[
            pl.BlockSpec((BLK, kdim), lambda i: (i, 0)),
            pl.BlockSpec((kdim, mdim), lambda i: (0, 0)),
        ],
        out_specs=pl.BlockSpec((BLK, mdim), lambda i: (i, 0)),
        out_shape=jax.ShapeDtypeStruct((n, mdim), jnp.float32),
    )(x, w)


def _colsum_body(y_ref, s_ref):
    s_ref[...] = jnp.sum(y_ref[...], axis=0, keepdims=True)[None]


def _colsum(y, n):
    nb = n // BLK
    return pl.pallas_call(
        _colsum_body,
        grid=(nb,),
        in_specs=[pl.BlockSpec((BLK, DH), lambda i: (i, 0))],
        out_specs=pl.BlockSpec((1, 1, DH), lambda i: (i, 0, 0)),
        out_shape=jax.ShapeDtypeStruct((nb, 1, DH), jnp.float32),
    )(y)


def _varpart_body(y_ref, mu_ref, s_ref):
    t = y_ref[...] - mu_ref[...]
    s_ref[...] = jnp.sum(t * t, axis=0, keepdims=True)[None]


def _varpart(y, mu, n):
    nb = n // BLK
    return pl.pallas_call(
        _varpart_body,
        grid=(nb,),
        in_specs=[
            pl.BlockSpec((BLK, DH), lambda i: (i, 0)),
            pl.BlockSpec((1, DH), lambda i: (0, 0)),
        ],
        out_specs=pl.BlockSpec((1, 1, DH), lambda i: (i, 0, 0)),
        out_shape=jax.ShapeDtypeStruct((nb, 1, DH), jnp.float32),
    )(y, mu)


def _bn_apply_body(y_ref, mu_ref, var_ref, g_ref, b_ref, o_ref, *, relu):
    o = (g_ref[...] * (y_ref[...] - mu_ref[...])
         / jnp.sqrt(var_ref[...] + 1e-5) + b_ref[...])
    if relu:
        o = jnp.maximum(o, 0.0)
    o_ref[...] = o


def _bn_apply(y, mu, var, g, b, n, relu):
    return pl.pallas_call(
        functools.partial(_bn_apply_body, relu=relu),
        grid=(n // BLK,),
        in_specs=[
            pl.BlockSpec((BLK, DH), lambda i: (i, 0)),
            pl.BlockSpec((1, DH), lambda i: (0, 0)),
            pl.BlockSpec((1, DH), lambda i: (0, 0)),
            pl.BlockSpec((1, DH), lambda i: (0, 0)),
            pl.BlockSpec((1, DH), lambda i: (0, 0)),
        ],
        out_specs=pl.BlockSpec((BLK, DH), lambda i: (i, 0)),
        out_shape=jax.ShapeDtypeStruct((n, DH), jnp.float32),
    )(y, mu, var, g.reshape(1, DH), b.reshape(1, DH))


def _batch_norm_opt(y, g, b, n, relu):
    nb = n // BLK
    s = _colsum(y, n)
    mu = (jnp.sum(s, axis=(0, 1)) / n).reshape(1, DH)
    vs = _varpart(y, mu, n)
    var = (jnp.sum(vs, axis=(0, 1)) / n).reshape(1, DH)
    return _bn_apply(y, mu, var, g, b, n, relu)


def _mp_dense_body(h_ref, agg_ref, deg_ref, w_ref, b_ref, o_ref, *, fuse_elu):
    h = h_ref[...]
    z = h + agg_ref[...] / deg_ref[...]
    z = jnp.dot(z, w_ref[...], preferred_element_type=jnp.float32) + b_ref[...]
    if fuse_elu:
        o_ref[...] = h + jnp.where(z > 0, z, (jnp.exp(z) - 1.0))
    else:
        o_ref[...] = z


def _mp_dense(h, agg, deg, w, b, n, fuse_elu=True):
    return pl.pallas_call(
        functools.partial(_mp_dense_body, fuse_elu=fuse_elu),
        grid=(n // BLK,),
        in_specs=[
            pl.BlockSpec((BLK, DH), lambda i: (i, 0)),
            pl.BlockSpec((BLK, DH), lambda i: (i, 0)),
            pl.BlockSpec((BLK, 1), lambda i: (i, 0)),
            pl.BlockSpec((DH, DH), lambda i: (0, 0)),
            pl.BlockSpec((1, DH), lambda i: (0, 0)),
        ],
        out_specs=pl.BlockSpec((BLK, DH), lambda i: (i, 0)),
        out_shape=jax.ShapeDtypeStruct((n, DH), jnp.float32),
    )(h, agg, deg, w, b.reshape(1, DH))


def _post_body(h_ref, hu_ref, w0, w1, w2, b0, b1, b2, y_ref, s_ref):
    x = h_ref[...] + hu_ref[...]

    def lin_elu(x, w, b):
        z = jnp.dot(x, w[...], preferred_element_type=jnp.float32) + b[...]
        return jnp.where(z > 0, z, (jnp.exp(z) - 1.0))

    x = lin_elu(x, w0, b0)
    x = lin_elu(x, w1, b1)
    x = lin_elu(x, w2, b2)
    y_ref[...] = x
    s_ref[...] = jnp.sum(x, axis=0, keepdims=True)[None]


def _post(h0, hu2, W_post, b_post, n):
    nb = n // BLK
    ws = [W_post[i] for i in range(3)]
    bs = [b_post[i].reshape(1, DH) for i in range(3)]
    return pl.pallas_call(
        _post_body,
        grid=(nb,),
        in_specs=[
            pl.BlockSpec((BLK, DH), lambda i: (i, 0)),
            pl.BlockSpec((BLK, DH), lambda i: (i, 0)),
        ]
        + [pl.BlockSpec((DH, DH), lambda i: (0, 0))] * 3
        + [pl.BlockSpec((1, DH), lambda i: (0, 0))] * 3,
        out_specs=[
            pl.BlockSpec((BLK, DH), lambda i: (i, 0)),
            pl.BlockSpec((1, 1, DH), lambda i: (i, 0, 0)),
        ],
        out_shape=[
            jax.ShapeDtypeStruct((n, DH), jnp.float32),
            jax.ShapeDtypeStruct((nb, 1, DH), jnp.float32),
        ],
    )(h0, hu2, *ws, *bs)


def _final_body(y_ref, mu_ref, var_ref, g_ref, b_ref, wc_ref, o_ref):
    x = (g_ref[...] * (y_ref[...] - mu_ref[...])
         / jnp.sqrt(var_ref[...] + 1e-5) + b_ref[...])
    o_ref[...] = jnp.dot(x, wc_ref[...], preferred_element_type=jnp.float32)


def _final(y, mu, var, g, b, wc_pad, n, ncls_pad):
    return pl.pallas_call(
        _final_body,
        grid=(n // BLK,),
        in_specs=[
            pl.BlockSpec((BLK, DH), lambda i: (i, 0)),
            pl.BlockSpec((1, DH), lambda i: (0, 0)),
            pl.BlockSpec((1, DH), lambda i: (0, 0)),
            pl.BlockSpec((1, DH), lambda i: (0, 0)),
            pl.BlockSpec((1, DH), lambda i: (0, 0)),
            pl.BlockSpec((DH, ncls_pad), lambda i: (0, 0)),
        ],
        out_specs=pl.BlockSpec((BLK, ncls_pad), lambda i: (i, 0)),
        out_shape=jax.ShapeDtypeStruct((n, ncls_pad), jnp.float32),
    )(y, mu, var, g.reshape(1, DH), b.reshape(1, DH), wc_pad)


_NC, _NS = 2, 16
_CE = 640


def _sc_seg_sum(h_lo, h_hi, src_e, dst_e, zeros_pad, n_pad, n_chunks, kmax):
    mesh = plsc.VectorSubcoreMesh(core_axis_name="c", subcore_axis_name="s")
    rsl = n_pad // _NS
    hw = DH // 2

    @functools.partial(
        pl.kernel, mesh=mesh,
        compiler_params=pltpu.CompilerParams(use_tc_tiling_on_sc=False),
        out_type=jax.ShapeDtypeStruct((n_pad, DH), jnp.float32),
        scratch_types=[
            pltpu.VMEM((_CE,), jnp.int32),
            pltpu.VMEM((_CE,), jnp.int32),
            pltpu.VMEM((_CE, hw), jnp.float32),
            pltpu.VMEM_SHARED((n_pad, hw), jnp.float32),
            pltpu.SemaphoreType.DMA,
        ],
    )
    def k(lo_hbm, hi_hbm, src_hbm, dst_hbm, z_hbm, out_hbm,
          sidx, didx, rows, acc, sem):
        c = lax.axis_index("c")
        s = lax.axis_index("s")
        pltpu.sync_copy(z_hbm.at[pl.ds(s * rsl, rsl)],
                        acc.at[pl.ds(s * rsl, rsl)])
        plsc.subcore_barrier()

        def loop(h_hbm):
            def body(kk, carry):
                chunk = s + _NS * kk

                @pl.when(chunk < n_chunks)
                def _():
                    rb = chunk * _CE
                    pltpu.sync_copy(src_hbm.at[pl.ds(rb, _CE)], sidx)
                    pltpu.sync_copy(dst_hbm.at[pl.ds(rb, _CE)], didx)
                    pltpu.async_copy(h_hbm.at[sidx], rows, sem).wait()
                    pltpu.sync_copy(rows, acc.at[didx], add=True)

                return carry

            lax.fori_loop(0, kmax, body, 0)

        @pl.when(c == 0)
        def _():
            loop(lo_hbm)

        @pl.when(c == 1)
        def _():
            loop(hi_hbm)

        plsc.subcore_barrier()
        pltpu.sync_copy(acc.at[pl.ds(s * rsl, rsl)],
                        out_hbm.at[pl.ds(s * rsl, rsl), pl.ds(c * hw, hw)])

    return k(h_lo, h_hi, src_e, dst_e, zeros_pad)


def _seg_sum(h, src, dst, n):
    return jax.ops.segment_sum(h[src], dst, num_segments=n)


def _deg(dst, n):
    deg = jax.ops.segment_sum(jnp.ones(dst.shape, jnp.float32), dst,
                              num_segments=n)
    return jnp.maximum(deg, 1.0).reshape(n, 1)


def _mp_dense2_body(h_ref, agg_ref, deg_ref, w_ref, b_ref, o_ref):
    h = h_ref[...]
    z = h + (agg_ref[0] + agg_ref[1]) / deg_ref[...]
    z = jnp.dot(z, w_ref[...], preferred_element_type=jnp.float32) + b_ref[...]
    o_ref[...] = h + jnp.where(z > 0, z, (jnp.exp(z) - 1.0))


def _mp_dense2(h, agg2, deg, w, b, n):
    return pl.pallas_call(
        _mp_dense2_body,
        grid=(n // BLK,),
        in_specs=[
            pl.BlockSpec((BLK, DH), lambda i: (i, 0)),
            pl.BlockSpec((2, BLK, DH), lambda i: (0, i, 0)),
            pl.BlockSpec((BLK, 1), lambda i: (i, 0)),
            pl.BlockSpec((DH, DH), lambda i: (0, 0)),
            pl.BlockSpec((1, DH), lambda i: (0, 0)),
        ],
        out_specs=pl.BlockSpec((BLK, DH), lambda i: (i, 0)),
        out_shape=jax.ShapeDtypeStruct((n, DH), jnp.float32),
    )(h, agg2, deg, w, b.reshape(1, DH))


def _mp_block_sc(h, src_e, dst_e, zeros_pad, Ws, bs, n, n_pad, n_chunks, kmax,
                 deg):
    for l in range(Ws.shape[0]):
        agg = _sc_seg_sum(h[:, :DH // 2], h[:, DH // 2:], src_e, dst_e,
                          zeros_pad, n_pad, n_chunks, kmax)
        h = _mp_dense(h, agg[:n, :], deg, Ws[l], bs[l], n)
    return h


def _mp_block_opt(h, src, dst, Ws, bs, n, exact=False):
    deg = _deg(dst, n)
    for l in range(Ws.shape[0]):
        agg = _seg_sum(h, src, dst, n)
        if exact:
            z = _mp_dense(h, agg, deg, Ws[l], bs[l], n, fuse_elu=False)
            h = h + jax.nn.elu(z)
        else:
            h = _mp_dense(h, agg, deg, Ws[l], bs[l], n)
    return h


def _push_up(h_coarse, idx, src_j, dst_i, n_fine):
    new_h = jnp.zeros((n_fine, DH), jnp.float32).at[idx].set(h_coarse)
    agg = jax.ops.segment_max(new_h[src_j], dst_i, num_segments=n_fine)
    return jnp.where(jnp.isfinite(agg), agg, 0.0)


def kernel(coord, feat, edge_index0, edge_index1, edge_index2, W_embed, g_embed,
           b_embed, W_mp0, b_mp0, W_mp1, b_mp1, W_bneck, b_bneck, w_score0,
           w_score1, W_up0, b_up0, W_up1, b_up1, W_post, b_post, g_final,
           b_final, W_cluster):
    del coord, edge_index2, W_bneck, b_bneck, w_score1, W_up0, b_up0

    y = feat @ W_embed
    mu0 = jnp.mean(y, axis=0, keepdims=True)
    var0 = jnp.var(y, axis=0, keepdims=True)
    h0 = jax.nn.relu(g_embed * (y - mu0) / jnp.sqrt(var0 + 1e-5) + b_embed)

    e0s, e0d = edge_index0[0], edge_index0[1]
    e1s, e1d = edge_index1[0], edge_index1[1]

    h0 = _mp_block_opt(h0, e0s, e0d, W_mp0, b_mp0, N0_, exact=True)

    v0, idx0 = jax.lax.top_k(h0 @ w_score0, N1_)
    h1 = h0[idx0] * jax.nn.sigmoid(v0)[:, None]

    deg1 = _deg(e1d, N1_)
    z1 = jnp.zeros((25088, DH // 2), jnp.float32)
    h1 = _mp_block_sc(h1, e1s, e1d, z1,
                      W_mp1, b_mp1, N1_, 25088, 625, 40, deg1)

    hu2 = _push_up(h1, idx0, e0d, e0s, N0_)
    hu2 = _mp_block_opt(hu2, e0s, e0d, W_up1, b_up1, N0_)

    y3, _ = _post(h0, hu2, W_post, b_post, N0_)
    mu3 = jnp.mean(y3, axis=0, keepdims=True)
    var3 = jnp.var(y3, axis=0, keepdims=True)
    x = g_final * (y3 - mu3) / jnp.sqrt(var3 + 1e-5) + b_final
    return x @ W_cluster
